# Initial kernel scaffold; baseline (speedup 1.0000x reference)
import jax
import jax.numpy as jnp
from jax.experimental import pallas as pl


def kernel(x, edge_index, W1, b1, W2, b2):
    raise NotImplementedError("write your pallas kernel here")


# speedup vs baseline: 3.5071x; 3.5071x over previous
import functools

import jax
import jax.numpy as jnp
from jax import lax
from jax.experimental import pallas as pl
from jax.experimental.pallas import tpu as pltpu
from jax.experimental.pallas import tpu_sc as plsc

NW = 32
NTILE = 16
B = 128
GROUP = 8


@functools.lru_cache(maxsize=None)
def _make_sc_agg(npad: int, width: int, cpw: int, with_deg: bool):
    rows_per_tile = npad // NTILE
    groups = cpw // GROUP
    mesh = plsc.VectorSubcoreMesh(core_axis_name="c", subcore_axis_name="s")

    out_type = [jax.ShapeDtypeStruct((2, npad, width), jnp.float32)]
    scratch = [
        pltpu.VMEM((GROUP, B), jnp.int32),
        pltpu.VMEM((GROUP, B), jnp.int32),
        pltpu.VMEM((B, width), jnp.float32),
        pltpu.VMEM((B, width), jnp.float32),
        pltpu.VMEM_SHARED((npad, width), jnp.float32),
        pltpu.SemaphoreType.DMA,
        pltpu.SemaphoreType.DMA,
    ]
    if with_deg:
        out_type.append(
            jax.ShapeDtypeStruct((NW * npad,), jnp.float32))
        scratch.append(pltpu.VMEM((npad,), jnp.float32))

    @functools.partial(
        pl.kernel, out_type=tuple(out_type), mesh=mesh,
        scratch_types=scratch,
        compiler_params=pltpu.CompilerParams(needs_layout_passes=False))
    def sc_agg(table_hbm, src_hbm, dst_hbm, out_hbm, *rest):
        if with_deg:
            (deg_hbm, src_g, dst_g, rows0, rows1, acc_sh, sem0, sem1,
             deg_v) = rest
        else:
            src_g, dst_g, rows0, rows1, acc_sh, sem0, sem1 = rest
        c = lax.axis_index("c")
        s = lax.axis_index("s")
        w = s * 2 + c

        zvec = jnp.zeros((16,), jnp.float32)

        def zrow(i, carry):
            for k in range(width // 16):
                rows0[i, pl.ds(k * 16, 16)] = zvec
                rows1[i, pl.ds(k * 16, 16)] = zvec
            return carry

        lax.fori_loop(0, B, zrow, 0)

        base = s * rows_per_tile
        nfull = rows_per_tile // B
        rem = rows_per_tile % B
        for t in range(nfull):
            pltpu.sync_copy(rows0, acc_sh.at[pl.ds(base + t * B, B)])
        if rem:
            pltpu.sync_copy(rows0.at[pl.ds(0, rem)],
                            acc_sh.at[pl.ds(base + nfull * B, rem)])

        if with_deg:
            def zdrow(i, carry):
                deg_v[pl.ds(i * 16, 16)] = zvec
                return carry

            lax.fori_loop(0, npad // 16, zdrow, 0)

        plsc.subcore_barrier()

        ones16 = jnp.ones((16,), jnp.float32)
        bufs = (rows0, rows1)
        sems = (sem0, sem1)

        def group_body(g, carry):
            gbase = (w * groups + g) * GROUP
            pltpu.sync_copy(src_hbm.at[pl.ds(gbase, GROUP)], src_g)
            pltpu.sync_copy(dst_hbm.at[pl.ds(gbase, GROUP)], dst_g)
            handles = [pltpu.async_copy(
                table_hbm.at[src_g.at[0]], bufs[0], sems[0])]
            for k in range(GROUP):
                if k + 1 < GROUP:
                    handles.append(pltpu.async_copy(
                        table_hbm.at[src_g.at[k + 1]],
                        bufs[(k + 1) % 2], sems[(k + 1) % 2]))
                if with_deg:
                    for t in range(B // 16):
                        dvec = dst_g[k, pl.ds(t * 16, 16)]
                        plsc.addupdate_scatter(deg_v, [dvec], ones16)
                handles[k].wait()
                pltpu.sync_copy(bufs[k % 2], acc_sh.at[dst_g.at[k]], add=True)
            return carry

        lax.fori_loop(0, groups, group_body, 0)

        plsc.subcore_barrier()

        pltpu.sync_copy(acc_sh.at[pl.ds(base, rows_per_tile)],
                        out_hbm.at[c, pl.ds(base, rows_per_tile)])
        if with_deg:
            pltpu.sync_copy(deg_v, deg_hbm.at[pl.ds(w * npad, npad)])

    return sc_agg


def _tc_layer1_body(n_real, acc_ref, x_ref, deg_ref, w_ref, b_ref,
                    h_ref, wdeg_ref):
    n = x_ref.shape[0]
    d = w_ref.shape[0]
    rows = n // 128
    agg = acc_ref[0] + acc_ref[1] + x_ref[...]
    deg2d = jnp.sum(deg_ref[...], axis=0)
    inv2d = 1.0 / (deg2d + 1.0)
    vid = (lax.broadcasted_iota(jnp.int32, (rows, 128), 0) * 128
           + lax.broadcasted_iota(jnp.int32, (rows, 128), 1))
    inv2d = jnp.where(vid < n_real, inv2d, 0.0)
    inv3 = inv2d[:, :, None]
    hn = (agg.reshape(rows, 128, d) * inv3).reshape(n, d)
    z = jnp.dot(hn, w_ref[...], preferred_element_type=jnp.float32) + b_ref[...]
    h1 = jnp.maximum(z, 0.0)
    nrm = jnp.sqrt(jnp.sum(h1 * h1, axis=1, keepdims=True))
    h = h1 / jnp.maximum(nrm, 1e-12)
    h = (h.reshape(rows, 128, d)
         * jnp.where(inv3 > 0.0, 1.0, 0.0)).reshape(n, d)
    h_ref[...] = h
    wdeg_ref[...] = (jnp.ones((rows, 128, d), jnp.float32)
                     * inv3).reshape(n, d)


def _tc_layer2_body(n_real, acc_ref, h_ref, wdeg_ref, w_ref, b_ref, out_ref):
    t = (acc_ref[0] + acc_ref[1] + h_ref[...]) * wdeg_ref[...]
    m = jnp.sum(t, axis=0, keepdims=True) * (1.0 / n_real)
    out_ref[...] = (
        jnp.dot(m, w_ref[...], preferred_element_type=jnp.float32) + b_ref[...]
    )


def kernel(x, edge_index, W1, b1, W2, b2):
    n, d = x.shape
    e = edge_index.shape[1]
    npad = (n // 128 + 1) * 128
    chunks = -(-e // B)
    cpw = -(-chunks // (NW * 8)) * 8
    chunks_pad = cpw * NW
    epad = chunks_pad * B

    src = edge_index[0]
    dst = edge_index[1]
    pad = epad - e
    fill = jnp.full((pad,), n, dtype=jnp.int32)
    src_p = jnp.concatenate([src, fill]).reshape(chunks_pad, B)
    dst_p = jnp.concatenate([dst, fill]).reshape(chunks_pad, B)

    x_pad = jnp.concatenate([x, jnp.zeros((npad - n, d), jnp.float32)], axis=0)

    acc1, deg_part = _make_sc_agg(npad, d, cpw, True)(x_pad, src_p, dst_p)
    deg_col = deg_part.reshape(NW, npad // 128, 128)

    h_pad, wdeg = pl.pallas_call(
        functools.partial(_tc_layer1_body, n),
        out_shape=(
            jax.ShapeDtypeStruct((npad, d), jnp.float32),
            jax.ShapeDtypeStruct((npad, d), jnp.float32),
        ),
    )(acc1, x_pad, deg_col, W1, b1.reshape(1, d))

    (acc2,) = _make_sc_agg(npad, d, cpw, False)(h_pad, src_p, dst_p)

    out = pl.pallas_call(
        functools.partial(_tc_layer2_body, float(n)),
        out_shape=jax.ShapeDtypeStruct((1, d), jnp.float32),
    )(acc2, h_pad, wdeg, W2, b2.reshape(1, d))

    return out


# speedup vs baseline: 10.9603x; 3.1252x over previous
import functools

import jax
import jax.numpy as jnp
from jax import lax
from jax.experimental import pallas as pl
from jax.experimental.pallas import tpu as pltpu
from jax.experimental.pallas import tpu_sc as plsc

NW = 32
NTILE = 16
B = 128
GROUP = 8


@functools.lru_cache(maxsize=None)
def _make_sc_agg(npad: int, width: int, cpw: int, with_deg: bool):
    rows_per_tile = npad // NTILE
    groups = cpw // GROUP
    mesh = plsc.VectorSubcoreMesh(core_axis_name="c", subcore_axis_name="s")

    out_type = [jax.ShapeDtypeStruct((2, npad, width), jnp.float32)]
    scratch = [
        pltpu.VMEM((GROUP, B), jnp.int32),
        pltpu.VMEM((GROUP, B), jnp.int32),
        pltpu.VMEM((B, width), jnp.float32),
        pltpu.VMEM((B, width), jnp.float32),
        pltpu.VMEM_SHARED((npad, width), jnp.float32),
        pltpu.SemaphoreType.DMA,
        pltpu.SemaphoreType.DMA,
    ]
    if with_deg:
        out_type.append(
            jax.ShapeDtypeStruct((NW * npad,), jnp.float32))
        scratch.append(pltpu.VMEM((npad,), jnp.float32))

    @functools.partial(
        pl.kernel, out_type=tuple(out_type), mesh=mesh,
        scratch_types=scratch,
        compiler_params=pltpu.CompilerParams(needs_layout_passes=False))
    def sc_agg(table_hbm, src_hbm, dst_hbm, out_hbm, *rest):
        if with_deg:
            (deg_hbm, src_g, dst_g, rows0, rows1, acc_sh, sem0, sem1,
             deg_v) = rest
        else:
            src_g, dst_g, rows0, rows1, acc_sh, sem0, sem1 = rest
        c = lax.axis_index("c")
        s = lax.axis_index("s")
        w = s * 2 + c

        zvec = jnp.zeros((16,), jnp.float32)

        def zrow(i, carry):
            for k in range(width // 16):
                rows0[i, pl.ds(k * 16, 16)] = zvec
                rows1[i, pl.ds(k * 16, 16)] = zvec
            return carry

        lax.fori_loop(0, B, zrow, 0)

        base = s * rows_per_tile
        nfull = rows_per_tile // B
        rem = rows_per_tile % B
        for t in range(nfull):
            pltpu.sync_copy(rows0, acc_sh.at[pl.ds(base + t * B, B)])
        if rem:
            pltpu.sync_copy(rows0.at[pl.ds(0, rem)],
                            acc_sh.at[pl.ds(base + nfull * B, rem)])

        if with_deg:
            def zdrow(i, carry):
                deg_v[pl.ds(i * 16, 16)] = zvec
                return carry

            lax.fori_loop(0, npad // 16, zdrow, 0)

        plsc.subcore_barrier()

        ones16 = jnp.ones((16,), jnp.float32)
        bufs = (rows0, rows1)
        sems = (sem0, sem1)

        def group_body(g, carry):
            gbase = (w * groups + g) * GROUP
            pltpu.sync_copy(src_hbm.at[pl.ds(gbase, GROUP)], src_g)
            pltpu.sync_copy(dst_hbm.at[pl.ds(gbase, GROUP)], dst_g)
            handles = [pltpu.async_copy(
                table_hbm.at[src_g.at[0]], bufs[0], sems[0])]
            for k in range(GROUP):
                if k + 1 < GROUP:
                    handles.append(pltpu.async_copy(
                        table_hbm.at[src_g.at[k + 1]],
                        bufs[(k + 1) % 2], sems[(k + 1) % 2]))
                if with_deg:
                    for t in range(B // 16):
                        dvec = dst_g[k, pl.ds(t * 16, 16)]
                        plsc.addupdate_scatter(deg_v, [dvec], ones16)
                handles[k].wait()
                pltpu.sync_copy(bufs[k % 2], acc_sh.at[dst_g.at[k]], add=True)
            return carry

        lax.fori_loop(0, groups, group_body, 0)

        plsc.subcore_barrier()

        pltpu.sync_copy(acc_sh.at[pl.ds(base, rows_per_tile)],
                        out_hbm.at[c, pl.ds(base, rows_per_tile)])
        if with_deg:
            pltpu.sync_copy(deg_v, deg_hbm.at[pl.ds(w * npad, npad)])

    return sc_agg


def _tc_layer1_body(n_real, acc_ref, x_ref, deg_ref, w_ref, b_ref,
                    h_ref, wdeg_ref):
    n = x_ref.shape[0]
    d = w_ref.shape[0]
    rows = n // 128
    agg = acc_ref[0] + acc_ref[1] + x_ref[...]
    deg2d = jnp.sum(deg_ref[...], axis=0)
    inv2d = 1.0 / (deg2d + 1.0)
    vid = (lax.broadcasted_iota(jnp.int32, (rows, 128), 0) * 128
           + lax.broadcasted_iota(jnp.int32, (rows, 128), 1))
    inv2d = jnp.where(vid < n_real, inv2d, 0.0)
    inv3 = inv2d[:, :, None]
    hn = (agg.reshape(rows, 128, d) * inv3).reshape(n, d)
    z = jnp.dot(hn, w_ref[...], preferred_element_type=jnp.float32) + b_ref[...]
    h1 = jnp.maximum(z, 0.0)
    nrm = jnp.sqrt(jnp.sum(h1 * h1, axis=1, keepdims=True))
    h = h1 / jnp.maximum(nrm, 1e-12)
    h = (h.reshape(rows, 128, d)
         * jnp.where(inv3 > 0.0, 1.0, 0.0)).reshape(n, d)
    h_ref[...] = h
    wdeg_ref[...] = (jnp.ones((rows, 128, d), jnp.float32)
                     * inv3).reshape(n, d)


def _tc_layer2_body(n_real, acc_ref, h_ref, wdeg_ref, w_ref, b_ref, out_ref):
    t = (acc_ref[0] + acc_ref[1] + h_ref[...]) * wdeg_ref[...]
    m = jnp.sum(t, axis=0, keepdims=True) * (1.0 / n_real)
    out_ref[...] = (
        jnp.dot(m, w_ref[...], preferred_element_type=jnp.float32) + b_ref[...]
    )


def kernel(x, edge_index, W1, b1, W2, b2):
    n, d = x.shape
    e = edge_index.shape[1]
    npad = (n // 128 + 1) * 128
    chunks = -(-e // B)
    cpw = -(-chunks // (NW * 8)) * 8
    chunks_pad = cpw * NW
    epad = chunks_pad * B

    src = edge_index[0]
    dst = edge_index[1]
    pad = epad - e
    fill = n + jnp.arange(pad, dtype=jnp.int32) % (npad - n)
    src_p = (jnp.concatenate([src, fill]).reshape(cpw, NW, B)
             .transpose(1, 0, 2).reshape(chunks_pad, B))
    dst_p = (jnp.concatenate([dst, fill]).reshape(cpw, NW, B)
             .transpose(1, 0, 2).reshape(chunks_pad, B))

    x_pad = jnp.concatenate([x, jnp.zeros((npad - n, d), jnp.float32)], axis=0)

    acc1, deg_part = _make_sc_agg(npad, d, cpw, True)(x_pad, src_p, dst_p)
    deg_col = deg_part.reshape(NW, npad // 128, 128)

    h_pad, wdeg = pl.pallas_call(
        functools.partial(_tc_layer1_body, n),
        out_shape=(
            jax.ShapeDtypeStruct((npad, d), jnp.float32),
            jax.ShapeDtypeStruct((npad, d), jnp.float32),
        ),
    )(acc1, x_pad, deg_col, W1, b1.reshape(1, d))

    (acc2,) = _make_sc_agg(npad, d, cpw, False)(h_pad, src_p, dst_p)

    out = pl.pallas_call(
        functools.partial(_tc_layer2_body, float(n)),
        out_shape=jax.ShapeDtypeStruct((1, d), jnp.float32),
    )(acc2, h_pad, wdeg, W2, b2.reshape(1, d))

    return out


# speedup vs baseline: 16.3800x; 1.4945x over previous
import functools

import jax
import jax.numpy as jnp
from jax import lax
from jax.experimental import pallas as pl
from jax.experimental.pallas import tpu as pltpu
from jax.experimental.pallas import tpu_sc as plsc

NW = 32
NTILE = 16
B = 128
GROUP = 8


@functools.lru_cache(maxsize=None)
def _make_sc_agg(npad: int, width: int, cpw: int, with_deg: bool):
    rows_per_tile = npad // NTILE
    groups = cpw // GROUP
    mesh = plsc.VectorSubcoreMesh(core_axis_name="c", subcore_axis_name="s")

    out_type = [jax.ShapeDtypeStruct((2, npad, width), jnp.float32)]
    scratch = [
        pltpu.VMEM((GROUP, B), jnp.int32),
        pltpu.VMEM((GROUP, B), jnp.int32),
        pltpu.VMEM((B, width), jnp.float32),
        pltpu.VMEM((B, width), jnp.float32),
        pltpu.VMEM_SHARED((npad, width), jnp.float32),
        pltpu.SemaphoreType.DMA,
        pltpu.SemaphoreType.DMA,
    ]
    if with_deg:
        out_type.append(
            jax.ShapeDtypeStruct((NW * npad,), jnp.float32))
        scratch.append(pltpu.VMEM((npad,), jnp.float32))

    @functools.partial(
        pl.kernel, out_type=tuple(out_type), mesh=mesh,
        scratch_types=scratch,
        compiler_params=pltpu.CompilerParams(needs_layout_passes=False))
    def sc_agg(table_hbm, src_hbm, dst_hbm, out_hbm, *rest):
        if with_deg:
            (deg_hbm, src_g, dst_g, rows0, rows1, acc_sh, sem0, sem1,
             deg_v) = rest
        else:
            src_g, dst_g, rows0, rows1, acc_sh, sem0, sem1 = rest
        c = lax.axis_index("c")
        s = lax.axis_index("s")
        w = s * 2 + c

        zvec = jnp.zeros((16,), jnp.float32)

        def zrow(i, carry):
            for k in range(width // 16):
                rows0[i, pl.ds(k * 16, 16)] = zvec
                rows1[i, pl.ds(k * 16, 16)] = zvec
            return carry

        lax.fori_loop(0, B, zrow, 0)

        base = s * rows_per_tile
        nfull = rows_per_tile // B
        rem = rows_per_tile % B
        for t in range(nfull):
            pltpu.sync_copy(rows0, acc_sh.at[pl.ds(base + t * B, B)])
        if rem:
            pltpu.sync_copy(rows0.at[pl.ds(0, rem)],
                            acc_sh.at[pl.ds(base + nfull * B, rem)])

        if with_deg:
            def zdrow(i, carry):
                deg_v[pl.ds(i * 16, 16)] = zvec
                return carry

            lax.fori_loop(0, npad // 16, zdrow, 0)

        plsc.subcore_barrier()

        ones16 = jnp.ones((16,), jnp.float32)
        bufs = (rows0, rows1)
        sems = (sem0, sem1)

        def group_body(g, carry):
            gbase = (w * groups + g) * GROUP
            pltpu.sync_copy(src_hbm.at[pl.ds(gbase, GROUP)], src_g)
            pltpu.sync_copy(dst_hbm.at[pl.ds(gbase, GROUP)], dst_g)
            handles = [pltpu.async_copy(
                table_hbm.at[src_g.at[0]], bufs[0], sems[0])]
            for k in range(GROUP):
                if k + 1 < GROUP:
                    handles.append(pltpu.async_copy(
                        table_hbm.at[src_g.at[k + 1]],
                        bufs[(k + 1) % 2], sems[(k + 1) % 2]))
                if with_deg:
                    for t in range(B // 16):
                        dvec = dst_g[k, pl.ds(t * 16, 16)]
                        plsc.addupdate_scatter(deg_v, [dvec], ones16)
                handles[k].wait()
                pltpu.sync_copy(bufs[k % 2], acc_sh.at[dst_g.at[k]], add=True)
            return carry

        lax.fori_loop(0, groups, group_body, 0)

        plsc.subcore_barrier()

        pltpu.sync_copy(acc_sh.at[pl.ds(base, rows_per_tile)],
                        out_hbm.at[c, pl.ds(base, rows_per_tile)])
        if with_deg:
            pltpu.sync_copy(deg_v, deg_hbm.at[pl.ds(w * npad, npad)])

    return sc_agg


def _tc_layer1_body(n_real, acc_ref, x_ref, deg_ref, w_ref, b_ref,
                    h_ref, inv_ref):
    n = x_ref.shape[0]
    d = w_ref.shape[0]
    rows = n // 128
    agg = acc_ref[0] + acc_ref[1] + x_ref[...]
    deg2d = jnp.sum(deg_ref[...], axis=0)
    inv2d = 1.0 / (deg2d + 1.0)
    vid = (lax.broadcasted_iota(jnp.int32, (rows, 128), 0) * 128
           + lax.broadcasted_iota(jnp.int32, (rows, 128), 1))
    inv2d = jnp.where(vid < n_real, inv2d, 0.0)
    inv3 = inv2d[:, :, None]
    hn = (agg.reshape(rows, 128, d) * inv3).reshape(n, d)
    z = jnp.dot(hn, w_ref[...], preferred_element_type=jnp.float32) + b_ref[...]
    h1 = jnp.maximum(z, 0.0)
    nrm = jnp.sqrt(jnp.sum(h1 * h1, axis=1, keepdims=True))
    h = h1 / jnp.maximum(nrm, 1e-12)
    h = (h.reshape(rows, 128, d)
         * jnp.where(inv3 > 0.0, 1.0, 0.0)).reshape(n, d)
    h_ref[...] = h
    inv_ref[...] = inv2d


@functools.lru_cache(maxsize=None)
def _make_sc_edge_u(npad: int, cpw: int):
    groups = cpw // GROUP
    mesh = plsc.VectorSubcoreMesh(core_axis_name="c", subcore_axis_name="s")

    @functools.partial(
        pl.kernel,
        out_type=jax.ShapeDtypeStruct((NW * npad,), jnp.float32),
        mesh=mesh,
        scratch_types=[
            pltpu.VMEM((GROUP, B), jnp.int32),
            pltpu.VMEM((GROUP, B), jnp.int32),
            pltpu.VMEM((npad,), jnp.float32),
            pltpu.VMEM((npad,), jnp.float32),
        ],
        compiler_params=pltpu.CompilerParams(needs_layout_passes=False))
    def sc_edge_u(inv_hbm, src_hbm, dst_hbm, u_hbm, src_g, dst_g,
                  inv_v, u_v):
        c = lax.axis_index("c")
        s = lax.axis_index("s")
        w = s * 2 + c

        zvec = jnp.zeros((16,), jnp.float32)

        def zrow(i, carry):
            u_v[pl.ds(i * 16, 16)] = zvec
            return carry

        lax.fori_loop(0, npad // 16, zrow, 0)
        pltpu.sync_copy(inv_hbm, inv_v)

        def group_body(g, carry):
            gbase = (w * groups + g) * GROUP
            pltpu.sync_copy(src_hbm.at[pl.ds(gbase, GROUP)], src_g)
            pltpu.sync_copy(dst_hbm.at[pl.ds(gbase, GROUP)], dst_g)
            for k in range(GROUP):
                for t in range(B // 16):
                    svec = src_g[k, pl.ds(t * 16, 16)]
                    dvec = dst_g[k, pl.ds(t * 16, 16)]
                    w16 = plsc.load_gather(inv_v, [dvec])
                    plsc.addupdate_scatter(u_v, [svec], w16)
            return carry

        lax.fori_loop(0, groups, group_body, 0)

        pltpu.sync_copy(u_v, u_hbm.at[pl.ds(w * npad, npad)])

    return sc_edge_u


def _tc_layer2_body(n_real, u_ref, inv_ref, h_ref, w_ref, b_ref, out_ref):
    rows, _ = inv_ref.shape
    d = w_ref.shape[0]
    coef = jnp.sum(u_ref[...], axis=0) + inv_ref[...]
    h3 = h_ref[...].reshape(rows, 128, d)
    s1 = jnp.sum(h3 * coef[:, :, None], axis=0)
    m = jnp.sum(s1, axis=0, keepdims=True) * (1.0 / n_real)
    out_ref[...] = (
        jnp.dot(m, w_ref[...], preferred_element_type=jnp.float32) + b_ref[...]
    )


def kernel(x, edge_index, W1, b1, W2, b2):
    n, d = x.shape
    e = edge_index.shape[1]
    npad = (n // 128 + 1) * 128
    chunks = -(-e // B)
    cpw = -(-chunks // (NW * 8)) * 8
    chunks_pad = cpw * NW
    epad = chunks_pad * B

    src = edge_index[0]
    dst = edge_index[1]
    pad = epad - e
    fill = n + jnp.arange(pad, dtype=jnp.int32) % (npad - n)
    src_p = (jnp.concatenate([src, fill]).reshape(cpw, NW, B)
             .transpose(1, 0, 2).reshape(chunks_pad, B))
    dst_p = (jnp.concatenate([dst, fill]).reshape(cpw, NW, B)
             .transpose(1, 0, 2).reshape(chunks_pad, B))

    x_pad = jnp.concatenate([x, jnp.zeros((npad - n, d), jnp.float32)], axis=0)

    acc1, deg_part = _make_sc_agg(npad, d, cpw, True)(x_pad, src_p, dst_p)
    deg_col = deg_part.reshape(NW, npad // 128, 128)

    h_pad, inv2d = pl.pallas_call(
        functools.partial(_tc_layer1_body, n),
        out_shape=(
            jax.ShapeDtypeStruct((npad, d), jnp.float32),
            jax.ShapeDtypeStruct((npad // 128, 128), jnp.float32),
        ),
    )(acc1, x_pad, deg_col, W1, b1.reshape(1, d))

    u_part = _make_sc_edge_u(npad, cpw)(inv2d.reshape(npad), src_p, dst_p)

    out = pl.pallas_call(
        functools.partial(_tc_layer2_body, float(n)),
        out_shape=jax.ShapeDtypeStruct((1, d), jnp.float32),
    )(u_part.reshape(NW, npad // 128, 128), inv2d, h_pad, W2,
      b2.reshape(1, d))

    return out
